# Initial kernel scaffold; baseline (speedup 1.0000x reference)
import jax
import jax.numpy as jnp
from jax.experimental import pallas as pl


def kernel(x, edge_index, W1, b1, W2, b2):
    raise NotImplementedError("write your pallas kernel here")


# speedup vs baseline: 42.2846x; 42.2846x over previous
import functools

import jax
import jax.numpy as jnp
from jax import lax
from jax.experimental import pallas as pl
from jax.experimental.pallas import tpu as pltpu
from jax.experimental.pallas import tpu_sc as plsc

N = 10000
E = 320000
D = 128
H = 24
C = 10

NP = 10240
NC, NS = 2, 16
NW = NC * NS
CH = 128
KC = 8
G = 10
CPW = KC * G
EPAD = NW * CPW * CH
ER = EPAD // CH
ZR = NP // NS
BR = 256


def _make_mesh():
    return plsc.VectorSubcoreMesh(
        core_axis_name="c", subcore_axis_name="s", num_cores=NC, num_subcores=NS
    )


def _zero_rows(buf, nrows, width):
    @pl.loop(0, nrows)
    def _(i):
        for k in range(width // 16):
            buf[i, pl.ds(k * 16, 16)] = jnp.zeros((16,), jnp.float32)


def _sc_degree(dst2d):
    mesh = _make_mesh()
    scratch = [
        pltpu.VMEM((KC, CH), jnp.int32),
        pltpu.VMEM((KC, CH), jnp.int32),
        pltpu.VMEM((CH, 16), jnp.float32),
        pltpu.VMEM((ZR, 16), jnp.float32),
        pltpu.VMEM_SHARED((NP, 16), jnp.float32),
        pltpu.SemaphoreType.DMA,
        pltpu.SemaphoreType.DMA,
        pltpu.SemaphoreType.DMA,
        pltpu.SemaphoreType.DMA,
    ]

    @functools.partial(
        pl.kernel,
        out_type=jax.ShapeDtypeStruct((NC, NP, 16), jnp.float32),
        mesh=mesh,
        scratch_types=scratch,
        compiler_params=pltpu.CompilerParams(use_tc_tiling_on_sc=False),
    )
    def deg_kernel(dst_hbm, out_hbm, didxA, didxB, ones, zbuf, acc,
                   semIA, semIB, semSA, semSB):
        cid = lax.axis_index("c")
        sid = lax.axis_index("s")
        wid = sid * NC + cid
        base = wid * CPW

        @pl.loop(0, CH)
        def _(i):
            ones[i, :] = jnp.ones((16,), jnp.float32)

        _zero_rows(zbuf, ZR, 16)
        pltpu.sync_copy(zbuf, acc.at[pl.ds(sid * ZR, ZR)])
        plsc.subcore_barrier()

        didx = (didxA, didxB)
        semI = (semIA, semIB)
        semS = (semSA, semSB)
        descI = [None, None]
        descS = [[], []]
        for g in range(G):
            b = g % 2
            if g == 0:
                pltpu.sync_copy(dst_hbm.at[pl.ds(base, KC)], didx[b])
            else:
                for d in descI[b]:
                    d.wait()
            for d in descS[1 - b]:
                d.wait()
            descS[1 - b] = []
            if g + 1 < G:
                descI[1 - b] = [
                    pltpu.async_copy(
                        dst_hbm.at[pl.ds(base + (g + 1) * KC, KC)],
                        didx[1 - b], semI[1 - b])
                ]
            for d in descS[b]:
                d.wait()
            descS[b] = []
            for j in range(KC):
                descS[b].append(
                    pltpu.async_copy(ones, acc.at[didx[b].at[j]],
                                     semS[b], add=True))
        for b in (0, 1):
            for d in descS[b]:
                d.wait()
        plsc.subcore_barrier()
        pltpu.sync_copy(acc.at[pl.ds(sid * ZR, ZR)],
                        out_hbm.at[cid, pl.ds(sid * ZR, ZR)])

    return deg_kernel(dst2d)


def _sc_aggregate(feat, src2d, dst2d, W):
    mesh = _make_mesh()
    scratch = [
        pltpu.VMEM((KC, CH), jnp.int32),
        pltpu.VMEM((KC, CH), jnp.int32),
        pltpu.VMEM((KC, CH), jnp.int32),
        pltpu.VMEM((KC, CH), jnp.int32),
        pltpu.VMEM((KC, CH, W), jnp.float32),
        pltpu.VMEM((KC, CH, W), jnp.float32),
        pltpu.VMEM((ZR, W), jnp.float32),
        pltpu.VMEM_SHARED((NP, W), jnp.float32),
        pltpu.SemaphoreType.DMA,
        pltpu.SemaphoreType.DMA,
        pltpu.SemaphoreType.DMA,
        pltpu.SemaphoreType.DMA,
        pltpu.SemaphoreType.DMA,
    ]

    @functools.partial(
        pl.kernel,
        out_type=jax.ShapeDtypeStruct((NC, NP, W), jnp.float32),
        mesh=mesh,
        scratch_types=scratch,
        compiler_params=pltpu.CompilerParams(use_tc_tiling_on_sc=False),
    )
    def agg_kernel(feat_hbm, src_hbm, dst_hbm, out_hbm,
                   sidxA, sidxB, didxA, didxB, rowsA, rowsB, zbuf, acc,
                   semIA, semIB, semG, semSA, semSB):
        cid = lax.axis_index("c")
        sid = lax.axis_index("s")
        wid = sid * NC + cid
        base = wid * CPW

        _zero_rows(zbuf, ZR, W)
        pltpu.sync_copy(zbuf, acc.at[pl.ds(sid * ZR, ZR)])
        plsc.subcore_barrier()

        sidx = (sidxA, sidxB)
        didx = (didxA, didxB)
        rows = (rowsA, rowsB)
        semI = (semIA, semIB)
        semS = (semSA, semSB)
        descI = [None, None]
        descS = [[], []]
        for g in range(G):
            b = g % 2
            if g == 0:
                pltpu.sync_copy(src_hbm.at[pl.ds(base, KC)], sidx[b])
                pltpu.sync_copy(dst_hbm.at[pl.ds(base, KC)], didx[b])
            else:
                for d in descI[b]:
                    d.wait()
            for d in descS[b]:
                d.wait()
            descS[b] = []
            descG = [
                pltpu.async_copy(feat_hbm.at[sidx[b].at[j]], rows[b].at[j],
                                 semG)
                for j in range(KC)
            ]
            for d in descS[1 - b]:
                d.wait()
            descS[1 - b] = []
            if g + 1 < G:
                descI[1 - b] = [
                    pltpu.async_copy(
                        src_hbm.at[pl.ds(base + (g + 1) * KC, KC)],
                        sidx[1 - b], semI[1 - b]),
                    pltpu.async_copy(
                        dst_hbm.at[pl.ds(base + (g + 1) * KC, KC)],
                        didx[1 - b], semI[1 - b]),
                ]
            for d in descG:
                d.wait()
            for j in range(KC):
                descS[b].append(
                    pltpu.async_copy(rows[b].at[j], acc.at[didx[b].at[j]],
                                     semS[b], add=True))
        for b in (0, 1):
            for d in descS[b]:
                d.wait()
        plsc.subcore_barrier()
        pltpu.sync_copy(acc.at[pl.ds(sid * ZR, ZR)],
                        out_hbm.at[cid, pl.ds(sid * ZR, ZR)])

    return agg_kernel(feat, src2d, dst2d)


def _dinv_of(deg2_block):
    s = deg2_block[0] + deg2_block[1]
    return lax.rsqrt(s[:, 0:1] + 1.0)


def _tc1(x_pad, W1p, deg2):
    def body(xr, wr, dr, out):
        dinv = _dinv_of(dr)
        h1 = jnp.dot(xr[...], wr[...], preferred_element_type=jnp.float32,
                     precision=lax.Precision.HIGHEST)
        out[...] = h1 * dinv

    return pl.pallas_call(
        body,
        grid=(NP // BR,),
        in_specs=[
            pl.BlockSpec((BR, D), lambda i: (i, 0)),
            pl.BlockSpec((D, 32), lambda i: (0, 0)),
            pl.BlockSpec((NC, BR, 16), lambda i: (0, i, 0)),
        ],
        out_specs=pl.BlockSpec((BR, 32), lambda i: (i, 0)),
        out_shape=jax.ShapeDtypeStruct((NP, 32), jnp.float32),
    )(x_pad, W1p, deg2)


def _tc2(agg1, hs1, deg2, W2p, b1p):
    def body(ar, hr, dr, wr, br, out):
        dinv = _dinv_of(dr)
        z = (ar[0] + ar[1] + hr[...]) * dinv + br[...]
        r = jnp.maximum(z, 0.0)
        h2 = jnp.dot(r, wr[...], preferred_element_type=jnp.float32,
                     precision=lax.Precision.HIGHEST)
        out[...] = h2 * dinv

    return pl.pallas_call(
        body,
        grid=(NP // BR,),
        in_specs=[
            pl.BlockSpec((NC, BR, 32), lambda i: (0, i, 0)),
            pl.BlockSpec((BR, 32), lambda i: (i, 0)),
            pl.BlockSpec((NC, BR, 16), lambda i: (0, i, 0)),
            pl.BlockSpec((32, 16), lambda i: (0, 0)),
            pl.BlockSpec((1, 32), lambda i: (0, 0)),
        ],
        out_specs=pl.BlockSpec((BR, 16), lambda i: (i, 0)),
        out_shape=jax.ShapeDtypeStruct((NP, 16), jnp.float32),
    )(agg1, hs1, deg2, W2p, b1p)


def _tc3(agg2, hs2, deg2, b2p):
    def body(ar, hr, dr, br, out):
        dinv = _dinv_of(dr)
        z = (ar[0] + ar[1] + hr[...]) * dinv + br[...]
        col = lax.broadcasted_iota(jnp.int32, (BR, 16), 1)
        mask = col < C
        zm = jnp.where(mask, z, -1e30)
        m = jnp.max(zm, axis=1, keepdims=True)
        e = jnp.where(mask, jnp.exp(z - m), 0.0)
        ssum = jnp.sum(e, axis=1, keepdims=True)
        out[...] = z - m - jnp.log(ssum)

    return pl.pallas_call(
        body,
        grid=(NP // BR,),
        in_specs=[
            pl.BlockSpec((NC, BR, 16), lambda i: (0, i, 0)),
            pl.BlockSpec((BR, 16), lambda i: (i, 0)),
            pl.BlockSpec((NC, BR, 16), lambda i: (0, i, 0)),
            pl.BlockSpec((1, 16), lambda i: (0, 0)),
        ],
        out_specs=pl.BlockSpec((BR, 16), lambda i: (i, 0)),
        out_shape=jax.ShapeDtypeStruct((NP, 16), jnp.float32),
    )(agg2, hs2, deg2, b2p)


def kernel(x, edge_index, W1, b1, W2, b2):
    src = edge_index[0].astype(jnp.int32)
    dst = edge_index[1].astype(jnp.int32)
    pad = jnp.arange(EPAD - E, dtype=jnp.int32)
    src2d = jnp.concatenate([src, pad % N]).reshape(ER, CH)
    dst2d = jnp.concatenate([dst, N + pad % (NP - N)]).reshape(ER, CH)
    x_pad = jnp.pad(x, ((0, NP - N), (0, 0)))
    W1p = jnp.pad(W1, ((0, 0), (0, 32 - H)))
    b1p = jnp.pad(b1, (0, 32 - H)).reshape(1, 32)
    W2p = jnp.pad(W2, ((0, 32 - H), (0, 16 - C)))
    b2p = jnp.pad(b2, (0, 16 - C)).reshape(1, 16)

    deg2 = _sc_degree(dst2d)
    hs1 = _tc1(x_pad, W1p, deg2)
    agg1 = _sc_aggregate(hs1, src2d, dst2d, 32)
    hs2 = _tc2(agg1, hs1, deg2, W2p, b1p)
    agg2 = _sc_aggregate(hs2, src2d, dst2d, 16)
    outp = _tc3(agg2, hs2, deg2, b2p)
    return outp[:N, :C]


# speedup vs baseline: 53.1080x; 1.2560x over previous
import functools

import jax
import jax.numpy as jnp
from jax import lax
from jax.experimental import pallas as pl
from jax.experimental.pallas import tpu as pltpu
from jax.experimental.pallas import tpu_sc as plsc

N = 10000
E = 320000
D = 128
H = 24
C = 10

NP = 10240
NC, NS = 2, 16
NW = NC * NS
CH = 128
KC = 6
G = 13
ER = E // CH
XROWS = ER - NW * KC * G
ZR = NP // NS
BR = 2048


def _make_mesh():
    return plsc.VectorSubcoreMesh(
        core_axis_name="c", subcore_axis_name="s", num_cores=NC, num_subcores=NS
    )


def _zero_rows(buf, nrows, width):
    @pl.loop(0, nrows)
    def _(i):
        for k in range(width // 16):
            buf[i, pl.ds(k * 16, 16)] = jnp.zeros((16,), jnp.float32)


def _sc_degree(dst2d):
    mesh = _make_mesh()
    scratch = [
        pltpu.VMEM((KC, CH), jnp.int32),
        pltpu.VMEM((KC, CH), jnp.int32),
        pltpu.VMEM((CH, 16), jnp.float32),
        pltpu.VMEM((ZR, 16), jnp.float32),
        pltpu.VMEM_SHARED((NP, 16), jnp.float32),
        pltpu.SemaphoreType.DMA,
        pltpu.SemaphoreType.DMA,
        pltpu.SemaphoreType.DMA,
        pltpu.SemaphoreType.DMA,
    ]

    @functools.partial(
        pl.kernel,
        out_type=jax.ShapeDtypeStruct((NC, NP, 16), jnp.float32),
        mesh=mesh,
        scratch_types=scratch,
        compiler_params=pltpu.CompilerParams(use_tc_tiling_on_sc=False),
    )
    def deg_kernel(dst_hbm, out_hbm, didxA, didxB, ones, zbuf, acc,
                   semIA, semIB, semSA, semSB):
        cid = lax.axis_index("c")
        sid = lax.axis_index("s")
        wid = sid * NC + cid
        base = KC * G * wid + jnp.minimum(wid, XROWS)

        @pl.loop(0, CH)
        def _(i):
            ones[i, :] = jnp.ones((16,), jnp.float32)

        _zero_rows(zbuf, ZR, 16)
        pltpu.sync_copy(zbuf, acc.at[pl.ds(sid * ZR, ZR)])
        plsc.subcore_barrier()

        didx = (didxA, didxB)
        semI = (semIA, semIB)
        semS = (semSA, semSB)
        descI = [None, None]
        descS = [[], []]
        for g in range(G):
            b = g % 2
            if g == 0:
                pltpu.sync_copy(dst_hbm.at[pl.ds(base, KC)], didx[b])
            else:
                for d in descI[b]:
                    d.wait()
            for d in descS[1 - b]:
                d.wait()
            descS[1 - b] = []
            if g + 1 < G:
                descI[1 - b] = [
                    pltpu.async_copy(
                        dst_hbm.at[pl.ds(base + (g + 1) * KC, KC)],
                        didx[1 - b], semI[1 - b])
                ]
            for d in descS[b]:
                d.wait()
            descS[b] = []
            for j in range(KC):
                descS[b].append(
                    pltpu.async_copy(ones, acc.at[didx[b].at[j]],
                                     semS[b], add=True))
        for b in (0, 1):
            for d in descS[b]:
                d.wait()

        @pl.when(wid < XROWS)
        def _():
            tb = KC * G * wid + wid + KC * G
            pltpu.sync_copy(dst_hbm.at[tb], didxA.at[0])
            pltpu.sync_copy(ones, acc.at[didxA.at[0]], add=True)

        plsc.subcore_barrier()
        pltpu.sync_copy(acc.at[pl.ds(sid * ZR, ZR)],
                        out_hbm.at[cid, pl.ds(sid * ZR, ZR)])

    return deg_kernel(dst2d)


def _sc_aggregate(feat, src2d, dst2d, W):
    mesh = _make_mesh()
    scratch = [
        pltpu.VMEM((KC, CH), jnp.int32),
        pltpu.VMEM((KC, CH), jnp.int32),
        pltpu.VMEM((KC, CH), jnp.int32),
        pltpu.VMEM((KC, CH), jnp.int32),
        pltpu.VMEM((KC, CH, W), jnp.float32),
        pltpu.VMEM((KC, CH, W), jnp.float32),
        pltpu.VMEM((ZR, W), jnp.float32),
        pltpu.VMEM_SHARED((NP, W), jnp.float32),
        pltpu.SemaphoreType.DMA,
        pltpu.SemaphoreType.DMA,
        pltpu.SemaphoreType.DMA,
        pltpu.SemaphoreType.DMA,
        pltpu.SemaphoreType.DMA,
    ]

    @functools.partial(
        pl.kernel,
        out_type=jax.ShapeDtypeStruct((NC, NP, W), jnp.float32),
        mesh=mesh,
        scratch_types=scratch,
        compiler_params=pltpu.CompilerParams(use_tc_tiling_on_sc=False),
    )
    def agg_kernel(feat_hbm, src_hbm, dst_hbm, out_hbm,
                   sidxA, sidxB, didxA, didxB, rowsA, rowsB, zbuf, acc,
                   semIA, semIB, semG, semSA, semSB):
        cid = lax.axis_index("c")
        sid = lax.axis_index("s")
        wid = sid * NC + cid
        base = KC * G * wid + jnp.minimum(wid, XROWS)

        _zero_rows(zbuf, ZR, W)
        pltpu.sync_copy(zbuf, acc.at[pl.ds(sid * ZR, ZR)])
        plsc.subcore_barrier()

        sidx = (sidxA, sidxB)
        didx = (didxA, didxB)
        rows = (rowsA, rowsB)
        semI = (semIA, semIB)
        semS = (semSA, semSB)
        descI = [None, None]
        descS = [[], []]
        for g in range(G):
            b = g % 2
            if g == 0:
                pltpu.sync_copy(src_hbm.at[pl.ds(base, KC)], sidx[b])
                pltpu.sync_copy(dst_hbm.at[pl.ds(base, KC)], didx[b])
            else:
                for d in descI[b]:
                    d.wait()
            for d in descS[b]:
                d.wait()
            descS[b] = []
            descG = [
                pltpu.async_copy(feat_hbm.at[sidx[b].at[j]], rows[b].at[j],
                                 semG)
                for j in range(KC)
            ]
            for d in descS[1 - b]:
                d.wait()
            descS[1 - b] = []
            if g + 1 < G:
                descI[1 - b] = [
                    pltpu.async_copy(
                        src_hbm.at[pl.ds(base + (g + 1) * KC, KC)],
                        sidx[1 - b], semI[1 - b]),
                    pltpu.async_copy(
                        dst_hbm.at[pl.ds(base + (g + 1) * KC, KC)],
                        didx[1 - b], semI[1 - b]),
                ]
            for d in descG:
                d.wait()
            for j in range(KC):
                descS[b].append(
                    pltpu.async_copy(rows[b].at[j], acc.at[didx[b].at[j]],
                                     semS[b], add=True))
        for b in (0, 1):
            for d in descS[b]:
                d.wait()

        @pl.when(wid < XROWS)
        def _():
            tb = KC * G * wid + wid + KC * G
            pltpu.sync_copy(src_hbm.at[tb], sidxA.at[0])
            pltpu.sync_copy(dst_hbm.at[tb], didxA.at[0])
            pltpu.sync_copy(feat_hbm.at[sidxA.at[0]], rowsA.at[0])
            pltpu.sync_copy(rowsA.at[0], acc.at[didxA.at[0]], add=True)

        plsc.subcore_barrier()
        pltpu.sync_copy(acc.at[pl.ds(sid * ZR, ZR)],
                        out_hbm.at[cid, pl.ds(sid * ZR, ZR)])

    return agg_kernel(feat, src2d, dst2d)


def _dinv_of(deg2_block):
    s = deg2_block[0] + deg2_block[1]
    return lax.rsqrt(s[:, 0:1] + 1.0)


def _tc1(x_pad, W1p, deg2):
    def body(xr, wr, dr, out):
        dinv = _dinv_of(dr)
        h1 = jnp.dot(xr[...], wr[...], preferred_element_type=jnp.float32,
                     precision=lax.Precision.HIGHEST)
        out[...] = h1 * dinv

    return pl.pallas_call(
        body,
        grid=(NP // BR,),
        in_specs=[
            pl.BlockSpec((BR, D), lambda i: (i, 0)),
            pl.BlockSpec((D, 32), lambda i: (0, 0)),
            pl.BlockSpec((NC, BR, 16), lambda i: (0, i, 0)),
        ],
        out_specs=pl.BlockSpec((BR, 32), lambda i: (i, 0)),
        out_shape=jax.ShapeDtypeStruct((NP, 32), jnp.float32),
    )(x_pad, W1p, deg2)


def _tc2(agg1, hs1, deg2, W2p, b1p):
    def body(ar, hr, dr, wr, br, out):
        dinv = _dinv_of(dr)
        z = (ar[0] + ar[1] + hr[...]) * dinv + br[...]
        r = jnp.maximum(z, 0.0)
        h2 = jnp.dot(r, wr[...], preferred_element_type=jnp.float32,
                     precision=lax.Precision.HIGHEST)
        out[...] = h2 * dinv

    return pl.pallas_call(
        body,
        grid=(NP // BR,),
        in_specs=[
            pl.BlockSpec((NC, BR, 32), lambda i: (0, i, 0)),
            pl.BlockSpec((BR, 32), lambda i: (i, 0)),
            pl.BlockSpec((NC, BR, 16), lambda i: (0, i, 0)),
            pl.BlockSpec((32, 16), lambda i: (0, 0)),
            pl.BlockSpec((1, 32), lambda i: (0, 0)),
        ],
        out_specs=pl.BlockSpec((BR, 16), lambda i: (i, 0)),
        out_shape=jax.ShapeDtypeStruct((NP, 16), jnp.float32),
    )(agg1, hs1, deg2, W2p, b1p)


def _tc3(agg2, hs2, deg2, b2p):
    def body(ar, hr, dr, br, out):
        dinv = _dinv_of(dr)
        z = (ar[0] + ar[1] + hr[...]) * dinv + br[...]
        col = lax.broadcasted_iota(jnp.int32, (BR, 16), 1)
        mask = col < C
        zm = jnp.where(mask, z, -1e30)
        m = jnp.max(zm, axis=1, keepdims=True)
        e = jnp.where(mask, jnp.exp(z - m), 0.0)
        ssum = jnp.sum(e, axis=1, keepdims=True)
        out[...] = z - m - jnp.log(ssum)

    return pl.pallas_call(
        body,
        grid=(NP // BR,),
        in_specs=[
            pl.BlockSpec((NC, BR, 16), lambda i: (0, i, 0)),
            pl.BlockSpec((BR, 16), lambda i: (i, 0)),
            pl.BlockSpec((NC, BR, 16), lambda i: (0, i, 0)),
            pl.BlockSpec((1, 16), lambda i: (0, 0)),
        ],
        out_specs=pl.BlockSpec((BR, 16), lambda i: (i, 0)),
        out_shape=jax.ShapeDtypeStruct((NP, 16), jnp.float32),
    )(agg2, hs2, deg2, b2p)


def kernel(x, edge_index, W1, b1, W2, b2):
    src2d = edge_index[0].astype(jnp.int32).reshape(ER, CH)
    dst2d = edge_index[1].astype(jnp.int32).reshape(ER, CH)
    x_pad = jnp.pad(x, ((0, NP - N), (0, 0)))
    W1p = jnp.pad(W1, ((0, 0), (0, 32 - H)))
    b1p = jnp.pad(b1, (0, 32 - H)).reshape(1, 32)
    W2p = jnp.pad(W2, ((0, 32 - H), (0, 16 - C)))
    b2p = jnp.pad(b2, (0, 16 - C)).reshape(1, 16)

    deg2 = _sc_degree(dst2d)
    hs1 = _tc1(x_pad, W1p, deg2)
    agg1 = _sc_aggregate(hs1, src2d, dst2d, 32)
    hs2 = _tc2(agg1, hs1, deg2, W2p, b1p)
    agg2 = _sc_aggregate(hs2, src2d, dst2d, 16)
    outp = _tc3(agg2, hs2, deg2, b2p)
    return outp[:N, :C]


# speedup vs baseline: 66.3360x; 1.2491x over previous
import functools

import jax
import jax.numpy as jnp
from jax import lax
from jax.experimental import pallas as pl
from jax.experimental.pallas import tpu as pltpu
from jax.experimental.pallas import tpu_sc as plsc

N = 10000
E = 320000
D = 128
H = 24
C = 10

NP = 10240
NC, NS = 2, 16
NW = NC * NS
CH = 128
KC = 6
G = 13
ER = E // CH
XROWS = ER - NW * KC * G
ZR = NP // NS
BR = 2048


def _make_mesh():
    return plsc.VectorSubcoreMesh(
        core_axis_name="c", subcore_axis_name="s", num_cores=NC, num_subcores=NS
    )


def _zero_rows(buf, nrows, width):
    @pl.loop(0, nrows)
    def _(i):
        for k in range(width // 16):
            buf[i, pl.ds(k * 16, 16)] = jnp.zeros((16,), jnp.float32)


def _sc_degree(dst2d):
    mesh = _make_mesh()
    scratch = [
        pltpu.VMEM((KC, CH), jnp.int32),
        pltpu.VMEM((KC, CH), jnp.int32),
        pltpu.VMEM((CH, 32), jnp.float32),
        pltpu.VMEM((ZR, 32), jnp.float32),
        pltpu.VMEM_SHARED((NP, 32), jnp.float32),
        pltpu.SemaphoreType.DMA,
        pltpu.SemaphoreType.DMA,
        pltpu.SemaphoreType.DMA,
        pltpu.SemaphoreType.DMA,
    ]

    @functools.partial(
        pl.kernel,
        out_type=jax.ShapeDtypeStruct((NC, NP, 32), jnp.float32),
        mesh=mesh,
        scratch_types=scratch,
        compiler_params=pltpu.CompilerParams(use_tc_tiling_on_sc=False),
    )
    def deg_kernel(dst_hbm, out_hbm, didxA, didxB, ones, zbuf, acc,
                   semIA, semIB, semSA, semSB):
        cid = lax.axis_index("c")
        sid = lax.axis_index("s")
        wid = sid * NC + cid
        base = KC * G * wid + jnp.minimum(wid, XROWS)

        @pl.loop(0, CH)
        def _(i):
            for k in range(2):
                ones[i, pl.ds(16 * k, 16)] = jnp.ones((16,), jnp.float32)

        _zero_rows(zbuf, ZR, 32)
        pltpu.sync_copy(zbuf, acc.at[pl.ds(sid * ZR, ZR)])
        plsc.subcore_barrier()

        didx = (didxA, didxB)
        semI = (semIA, semIB)
        semS = (semSA, semSB)
        descI = [None, None]
        descS = [[], []]
        for g in range(G):
            b = g % 2
            if g == 0:
                pltpu.sync_copy(dst_hbm.at[pl.ds(base, KC)], didx[b])
            else:
                for d in descI[b]:
                    d.wait()
            for d in descS[1 - b]:
                d.wait()
            descS[1 - b] = []
            if g + 1 < G:
                descI[1 - b] = [
                    pltpu.async_copy(
                        dst_hbm.at[pl.ds(base + (g + 1) * KC, KC)],
                        didx[1 - b], semI[1 - b])
                ]
            for d in descS[b]:
                d.wait()
            descS[b] = []
            for j in range(KC):
                descS[b].append(
                    pltpu.async_copy(ones, acc.at[didx[b].at[j]],
                                     semS[b], add=True))
        for b in (0, 1):
            for d in descS[b]:
                d.wait()

        @pl.when(wid < XROWS)
        def _():
            tb = KC * G * wid + wid + KC * G
            pltpu.sync_copy(dst_hbm.at[tb], didxA.at[0])
            pltpu.sync_copy(ones, acc.at[didxA.at[0]], add=True)

        plsc.subcore_barrier()
        pltpu.sync_copy(acc.at[pl.ds(sid * ZR, ZR)],
                        out_hbm.at[cid, pl.ds(sid * ZR, ZR)])

    return deg_kernel(dst2d)


def _sc_aggregate(feat, src2d, dst2d, W):
    mesh = _make_mesh()
    scratch = [
        pltpu.VMEM((KC, CH), jnp.int32),
        pltpu.VMEM((KC, CH), jnp.int32),
        pltpu.VMEM((KC, CH), jnp.int32),
        pltpu.VMEM((KC, CH), jnp.int32),
        pltpu.VMEM((KC, CH, W), jnp.float32),
        pltpu.VMEM((KC, CH, W), jnp.float32),
        pltpu.VMEM((ZR, W), jnp.float32),
        pltpu.VMEM_SHARED((NP, W), jnp.float32),
        pltpu.SemaphoreType.DMA,
        pltpu.SemaphoreType.DMA,
        pltpu.SemaphoreType.DMA,
        pltpu.SemaphoreType.DMA,
        pltpu.SemaphoreType.DMA,
    ]

    @functools.partial(
        pl.kernel,
        out_type=jax.ShapeDtypeStruct((NC, NP, W), jnp.float32),
        mesh=mesh,
        scratch_types=scratch,
        compiler_params=pltpu.CompilerParams(use_tc_tiling_on_sc=False),
    )
    def agg_kernel(feat_hbm, src_hbm, dst_hbm, out_hbm,
                   sidxA, sidxB, didxA, didxB, rowsA, rowsB, zbuf, acc,
                   semIA, semIB, semG, semSA, semSB):
        cid = lax.axis_index("c")
        sid = lax.axis_index("s")
        wid = sid * NC + cid
        base = KC * G * wid + jnp.minimum(wid, XROWS)

        _zero_rows(zbuf, ZR, W)
        pltpu.sync_copy(zbuf, acc.at[pl.ds(sid * ZR, ZR)])
        plsc.subcore_barrier()

        sidx = (sidxA, sidxB)
        didx = (didxA, didxB)
        rows = (rowsA, rowsB)
        semI = (semIA, semIB)
        semS = (semSA, semSB)
        descI = [None, None]
        descS = [[], []]
        for g in range(G):
            b = g % 2
            if g == 0:
                pltpu.sync_copy(src_hbm.at[pl.ds(base, KC)], sidx[b])
                pltpu.sync_copy(dst_hbm.at[pl.ds(base, KC)], didx[b])
            else:
                for d in descI[b]:
                    d.wait()
            for d in descS[b]:
                d.wait()
            descS[b] = []
            descG = [
                pltpu.async_copy(feat_hbm.at[sidx[b].at[j]], rows[b].at[j],
                                 semG)
                for j in range(KC)
            ]
            for d in descS[1 - b]:
                d.wait()
            descS[1 - b] = []
            if g + 1 < G:
                descI[1 - b] = [
                    pltpu.async_copy(
                        src_hbm.at[pl.ds(base + (g + 1) * KC, KC)],
                        sidx[1 - b], semI[1 - b]),
                    pltpu.async_copy(
                        dst_hbm.at[pl.ds(base + (g + 1) * KC, KC)],
                        didx[1 - b], semI[1 - b]),
                ]
            for d in descG:
                d.wait()
            for j in range(KC):
                descS[b].append(
                    pltpu.async_copy(rows[b].at[j], acc.at[didx[b].at[j]],
                                     semS[b], add=True))
        for b in (0, 1):
            for d in descS[b]:
                d.wait()

        @pl.when(wid < XROWS)
        def _():
            tb = KC * G * wid + wid + KC * G
            pltpu.sync_copy(src_hbm.at[tb], sidxA.at[0])
            pltpu.sync_copy(dst_hbm.at[tb], didxA.at[0])
            pltpu.sync_copy(feat_hbm.at[sidxA.at[0]], rowsA.at[0])
            pltpu.sync_copy(rowsA.at[0], acc.at[didxA.at[0]], add=True)

        plsc.subcore_barrier()
        pltpu.sync_copy(acc.at[pl.ds(sid * ZR, ZR)],
                        out_hbm.at[cid, pl.ds(sid * ZR, ZR)])

    return agg_kernel(feat, src2d, dst2d)


def _tc_split(edge_index):
    def body(er, sr, dr):
        sr[...] = er[0]
        dr[...] = er[1]

    return pl.pallas_call(
        body,
        out_shape=[
            jax.ShapeDtypeStruct((E,), jnp.int32),
            jax.ShapeDtypeStruct((E,), jnp.int32),
        ],
    )(edge_index)


PB = BR // 4


def _dinvp_of(dr):
    return lax.rsqrt(dr[0] + dr[1] + 1.0)


def _tc1(xp, W1BD, deg2r):
    def body(xr, wr, dr, out):
        dinvp = _dinvp_of(dr)
        h1p = jnp.dot(xr[...], wr[...], preferred_element_type=jnp.float32,
                      precision=lax.Precision.HIGHEST)
        out[...] = h1p * dinvp

    return pl.pallas_call(
        body,
        grid=(NP // BR,),
        in_specs=[
            pl.BlockSpec((PB, 4 * D), lambda i: (i, 0)),
            pl.BlockSpec((4 * D, 128), lambda i: (0, 0)),
            pl.BlockSpec((NC, PB, 128), lambda i: (0, i, 0)),
        ],
        out_specs=pl.BlockSpec((PB, 128), lambda i: (i, 0)),
        out_shape=jax.ShapeDtypeStruct((NP // 4, 128), jnp.float32),
    )(xp, W1BD, deg2r)


def _tc2(agg1r, hs1p, deg2r, W2BD, b1p):
    def body(ar, hr, dr, wr, br, out):
        dinvp = _dinvp_of(dr)
        z = (ar[0] + ar[1] + hr[...]) * dinvp + br[...]
        r = jnp.maximum(z, 0.0)
        h2p = jnp.dot(r, wr[...], preferred_element_type=jnp.float32,
                      precision=lax.Precision.HIGHEST)
        out[...] = h2p * dinvp

    return pl.pallas_call(
        body,
        grid=(NP // BR,),
        in_specs=[
            pl.BlockSpec((NC, PB, 128), lambda i: (0, i, 0)),
            pl.BlockSpec((PB, 128), lambda i: (i, 0)),
            pl.BlockSpec((NC, PB, 128), lambda i: (0, i, 0)),
            pl.BlockSpec((128, 128), lambda i: (0, 0)),
            pl.BlockSpec((1, 128), lambda i: (0, 0)),
        ],
        out_specs=pl.BlockSpec((PB, 128), lambda i: (i, 0)),
        out_shape=jax.ShapeDtypeStruct((NP // 4, 128), jnp.float32),
    )(agg1r, hs1p, deg2r, W2BD, b1p)


def _tc3(agg2r, hs2p, deg2r, b2p):
    def body(ar, hr, dr, br, out):
        dinvp = _dinvp_of(dr)
        z = (ar[0] + ar[1] + hr[...]) * dinvp + br[...]
        col = lax.broadcasted_iota(jnp.int32, (PB, 32), 1)
        mask = col < C
        for k in range(4):
            zk = z[:, 32 * k:32 * k + 32]
            zm = jnp.where(mask, zk, -1e30)
            m = jnp.max(zm, axis=1, keepdims=True)
            e = jnp.where(mask, jnp.exp(zk - m), 0.0)
            ssum = jnp.sum(e, axis=1, keepdims=True)
            out[:, 32 * k:32 * k + 32] = zk - m - jnp.log(ssum)

    return pl.pallas_call(
        body,
        grid=(NP // BR,),
        in_specs=[
            pl.BlockSpec((NC, PB, 128), lambda i: (0, i, 0)),
            pl.BlockSpec((PB, 128), lambda i: (i, 0)),
            pl.BlockSpec((NC, PB, 128), lambda i: (0, i, 0)),
            pl.BlockSpec((1, 128), lambda i: (0, 0)),
        ],
        out_specs=pl.BlockSpec((PB, 128), lambda i: (i, 0)),
        out_shape=jax.ShapeDtypeStruct((NP // 4, 128), jnp.float32),
    )(agg2r, hs2p, deg2r, b2p)


def kernel(x, edge_index, W1, b1, W2, b2):
    src1d, dst1d = _tc_split(edge_index.astype(jnp.int32))
    src2d = jnp.reshape(src1d, (ER, CH))
    dst2d = jnp.reshape(dst1d, (ER, CH))
    xp = jnp.reshape(jnp.pad(x, ((0, NP - N), (0, 0))), (NP // 4, 4 * D))
    eye4 = jnp.eye(4, dtype=jnp.float32)
    W1BD = jnp.kron(eye4, jnp.pad(W1, ((0, 0), (0, 32 - H))))
    W2BD = jnp.kron(eye4, jnp.pad(W2, ((0, 32 - H), (0, 32 - C))))
    b1p = jnp.tile(jnp.pad(b1, (0, 32 - H)), 4).reshape(1, 128)
    b2p = jnp.tile(jnp.pad(b2, (0, 32 - C)), 4).reshape(1, 128)

    deg2 = _sc_degree(dst2d)
    deg2r = jnp.reshape(deg2, (NC, NP // 4, 128))
    hs1p = _tc1(xp, W1BD, deg2r)
    agg1 = _sc_aggregate(jnp.reshape(hs1p, (NP, 32)), src2d, dst2d, 32)
    agg1r = jnp.reshape(agg1, (NC, NP // 4, 128))
    hs2p = _tc2(agg1r, hs1p, deg2r, W2BD, b1p)
    agg2 = _sc_aggregate(jnp.reshape(hs2p, (NP, 32)), src2d, dst2d, 32)
    agg2r = jnp.reshape(agg2, (NC, NP // 4, 128))
    outp = _tc3(agg2r, hs2p, deg2r, b2p)
    return jnp.reshape(outp, (NP, 32))[:N, :C]


# speedup vs baseline: 73.3325x; 1.1055x over previous
import functools

import jax
import jax.numpy as jnp
from jax import lax
from jax.experimental import pallas as pl
from jax.experimental.pallas import tpu as pltpu
from jax.experimental.pallas import tpu_sc as plsc

N = 10000
E = 320000
D = 128
H = 24
C = 10

NP = 10240
NC, NS = 2, 16
NW = NC * NS
CH = 128
KC = 6
G = 13
ER = E // CH
XROWS = ER - NW * KC * G
ZR = NP // NS
BR = 2048


def _make_mesh():
    return plsc.VectorSubcoreMesh(
        core_axis_name="c", subcore_axis_name="s", num_cores=NC, num_subcores=NS
    )


ZB = 64


def _zero_acc(zbuf, acc, sid, sem):
    @pl.loop(0, ZB)
    def _(i):
        for k in range(2):
            zbuf[i, pl.ds(16 * k, 16)] = jnp.zeros((16,), jnp.float32)

    descs = [
        pltpu.async_copy(zbuf, acc.at[pl.ds(sid * ZR + ZB * t, ZB)], sem)
        for t in range(ZR // ZB)
    ]
    for d in descs:
        d.wait()


def _sc_degree(dst2d):
    mesh = _make_mesh()
    scratch = [
        pltpu.VMEM((KC, CH), jnp.int32),
        pltpu.VMEM((KC, CH), jnp.int32),
        pltpu.VMEM((CH, 32), jnp.float32),
        pltpu.VMEM((ZB, 32), jnp.float32),
        pltpu.VMEM_SHARED((NP, 32), jnp.float32),
        pltpu.SemaphoreType.DMA,
        pltpu.SemaphoreType.DMA,
        pltpu.SemaphoreType.DMA,
        pltpu.SemaphoreType.DMA,
    ]

    @functools.partial(
        pl.kernel,
        out_type=jax.ShapeDtypeStruct((NC, NP, 32), jnp.float32),
        mesh=mesh,
        scratch_types=scratch,
        compiler_params=pltpu.CompilerParams(use_tc_tiling_on_sc=False),
    )
    def deg_kernel(dst_hbm, out_hbm, didxA, didxB, ones, zbuf, acc,
                   semIA, semIB, semSA, semSB):
        cid = lax.axis_index("c")
        sid = lax.axis_index("s")
        wid = sid * NC + cid
        base = KC * G * wid + jnp.minimum(wid, XROWS)

        @pl.loop(0, CH)
        def _(i):
            for k in range(2):
                ones[i, pl.ds(16 * k, 16)] = jnp.ones((16,), jnp.float32)

        _zero_acc(zbuf, acc, sid, semIA)
        plsc.subcore_barrier()

        didx = (didxA, didxB)
        semI = (semIA, semIB)
        semS = (semSA, semSB)
        descI = [None, None]
        descS = [[], []]
        for g in range(G):
            b = g % 2
            if g == 0:
                pltpu.sync_copy(dst_hbm.at[pl.ds(base, KC)], didx[b])
            else:
                for d in descI[b]:
                    d.wait()
            for d in descS[1 - b]:
                d.wait()
            descS[1 - b] = []
            if g + 1 < G:
                descI[1 - b] = [
                    pltpu.async_copy(
                        dst_hbm.at[pl.ds(base + (g + 1) * KC, KC)],
                        didx[1 - b], semI[1 - b])
                ]
            for d in descS[b]:
                d.wait()
            descS[b] = []
            for j in range(KC):
                descS[b].append(
                    pltpu.async_copy(ones, acc.at[didx[b].at[j]],
                                     semS[b], add=True))
        for b in (0, 1):
            for d in descS[b]:
                d.wait()

        @pl.when(wid < XROWS)
        def _():
            tb = KC * G * wid + wid + KC * G
            pltpu.sync_copy(dst_hbm.at[tb], didxA.at[0])
            pltpu.sync_copy(ones, acc.at[didxA.at[0]], add=True)

        plsc.subcore_barrier()
        pltpu.sync_copy(acc.at[pl.ds(sid * ZR, ZR)],
                        out_hbm.at[cid, pl.ds(sid * ZR, ZR)])

    return deg_kernel(dst2d)


def _sc_aggregate(feat, src2d, dst2d, W):
    mesh = _make_mesh()
    scratch = [
        pltpu.VMEM((KC, CH), jnp.int32),
        pltpu.VMEM((KC, CH), jnp.int32),
        pltpu.VMEM((KC, CH), jnp.int32),
        pltpu.VMEM((KC, CH), jnp.int32),
        pltpu.VMEM((KC, CH, W), jnp.float32),
        pltpu.VMEM((KC, CH, W), jnp.float32),
        pltpu.VMEM((ZB, W), jnp.float32),
        pltpu.VMEM_SHARED((NP, W), jnp.float32),
        pltpu.SemaphoreType.DMA,
        pltpu.SemaphoreType.DMA,
        pltpu.SemaphoreType.DMA,
        pltpu.SemaphoreType.DMA,
    ] + [pltpu.SemaphoreType.DMA] * KC

    @functools.partial(
        pl.kernel,
        out_type=jax.ShapeDtypeStruct((NC, NP, W), jnp.float32),
        mesh=mesh,
        scratch_types=scratch,
        compiler_params=pltpu.CompilerParams(use_tc_tiling_on_sc=False),
    )
    def agg_kernel(feat_hbm, src_hbm, dst_hbm, out_hbm,
                   sidxA, sidxB, didxA, didxB, rowsA, rowsB, zbuf, acc,
                   semIA, semIB, semSA, semSB, *semGs):
        cid = lax.axis_index("c")
        sid = lax.axis_index("s")
        wid = sid * NC + cid
        base = KC * G * wid + jnp.minimum(wid, XROWS)

        _zero_acc(zbuf, acc, sid, semIA)
        plsc.subcore_barrier()

        sidx = (sidxA, sidxB)
        didx = (didxA, didxB)
        rows = (rowsA, rowsB)
        semI = (semIA, semIB)
        semS = (semSA, semSB)
        descI = [None, None]
        descS = [[], []]
        for g in range(G):
            b = g % 2
            if g == 0:
                pltpu.sync_copy(src_hbm.at[pl.ds(base, KC)], sidx[b])
                pltpu.sync_copy(dst_hbm.at[pl.ds(base, KC)], didx[b])
            else:
                for d in descI[b]:
                    d.wait()
            for d in descS[b]:
                d.wait()
            descS[b] = []
            descG = [
                pltpu.async_copy(feat_hbm.at[sidx[b].at[j]], rows[b].at[j],
                                 semGs[j])
                for j in range(KC)
            ]
            for d in descS[1 - b]:
                d.wait()
            descS[1 - b] = []
            if g + 1 < G:
                descI[1 - b] = [
                    pltpu.async_copy(
                        src_hbm.at[pl.ds(base + (g + 1) * KC, KC)],
                        sidx[1 - b], semI[1 - b]),
                    pltpu.async_copy(
                        dst_hbm.at[pl.ds(base + (g + 1) * KC, KC)],
                        didx[1 - b], semI[1 - b]),
                ]
            for j in range(KC):
                descG[j].wait()
                descS[b].append(
                    pltpu.async_copy(rows[b].at[j], acc.at[didx[b].at[j]],
                                     semS[b], add=True))
        for b in (0, 1):
            for d in descS[b]:
                d.wait()

        @pl.when(wid < XROWS)
        def _():
            tb = KC * G * wid + wid + KC * G
            pltpu.sync_copy(src_hbm.at[tb], sidxA.at[0])
            pltpu.sync_copy(dst_hbm.at[tb], didxA.at[0])
            pltpu.sync_copy(feat_hbm.at[sidxA.at[0]], rowsA.at[0])
            pltpu.sync_copy(rowsA.at[0], acc.at[didxA.at[0]], add=True)

        plsc.subcore_barrier()
        pltpu.sync_copy(acc.at[pl.ds(sid * ZR, ZR)],
                        out_hbm.at[cid, pl.ds(sid * ZR, ZR)])

    return agg_kernel(feat, src2d, dst2d)


def _tc_split(edge_index):
    def body(er, sr, dr):
        sr[...] = er[0]
        dr[...] = er[1]

    return pl.pallas_call(
        body,
        out_shape=[
            jax.ShapeDtypeStruct((E,), jnp.int32),
            jax.ShapeDtypeStruct((E,), jnp.int32),
        ],
    )(edge_index)


PB = BR // 4


def _dinvp_of(dr):
    return lax.rsqrt(dr[0] + dr[1] + 1.0)


def _tc1(xp, W1BD, deg2r):
    def body(xr, wr, dr, out):
        dinvp = _dinvp_of(dr)
        h1p = jnp.dot(xr[...], wr[...], preferred_element_type=jnp.float32)
        out[...] = h1p * dinvp

    return pl.pallas_call(
        body,
        grid=(NP // BR,),
        in_specs=[
            pl.BlockSpec((PB, 4 * D), lambda i: (i, 0)),
            pl.BlockSpec((4 * D, 128), lambda i: (0, 0)),
            pl.BlockSpec((NC, PB, 128), lambda i: (0, i, 0)),
        ],
        out_specs=pl.BlockSpec((PB, 128), lambda i: (i, 0)),
        out_shape=jax.ShapeDtypeStruct((NP // 4, 128), jnp.float32),
    )(xp, W1BD, deg2r)


def _tc2(agg1r, hs1p, deg2r, W2BD, b1p):
    def body(ar, hr, dr, wr, br, out):
        dinvp = _dinvp_of(dr)
        z = (ar[0] + ar[1] + hr[...]) * dinvp + br[...]
        r = jnp.maximum(z, 0.0)
        h2p = jnp.dot(r, wr[...], preferred_element_type=jnp.float32)
        out[...] = h2p * dinvp

    return pl.pallas_call(
        body,
        grid=(NP // BR,),
        in_specs=[
            pl.BlockSpec((NC, PB, 128), lambda i: (0, i, 0)),
            pl.BlockSpec((PB, 128), lambda i: (i, 0)),
            pl.BlockSpec((NC, PB, 128), lambda i: (0, i, 0)),
            pl.BlockSpec((128, 128), lambda i: (0, 0)),
            pl.BlockSpec((1, 128), lambda i: (0, 0)),
        ],
        out_specs=pl.BlockSpec((PB, 128), lambda i: (i, 0)),
        out_shape=jax.ShapeDtypeStruct((NP // 4, 128), jnp.float32),
    )(agg1r, hs1p, deg2r, W2BD, b1p)


def _tc3(agg2r, hs2p, deg2r, b2p):
    def body(ar, hr, dr, br, out):
        dinvp = _dinvp_of(dr)
        z = (ar[0] + ar[1] + hr[...]) * dinvp + br[...]
        col = lax.broadcasted_iota(jnp.int32, (PB, 32), 1)
        mask = col < C
        for k in range(4):
            zk = z[:, 32 * k:32 * k + 32]
            zm = jnp.where(mask, zk, -1e30)
            m = jnp.max(zm, axis=1, keepdims=True)
            e = jnp.where(mask, jnp.exp(zk - m), 0.0)
            ssum = jnp.sum(e, axis=1, keepdims=True)
            out[:, 32 * k:32 * k + 32] = zk - m - jnp.log(ssum)

    return pl.pallas_call(
        body,
        grid=(NP // BR,),
        in_specs=[
            pl.BlockSpec((NC, PB, 128), lambda i: (0, i, 0)),
            pl.BlockSpec((PB, 128), lambda i: (i, 0)),
            pl.BlockSpec((NC, PB, 128), lambda i: (0, i, 0)),
            pl.BlockSpec((1, 128), lambda i: (0, 0)),
        ],
        out_specs=pl.BlockSpec((PB, 128), lambda i: (i, 0)),
        out_shape=jax.ShapeDtypeStruct((NP // 4, 128), jnp.float32),
    )(agg2r, hs2p, deg2r, b2p)


def kernel(x, edge_index, W1, b1, W2, b2):
    src1d, dst1d = _tc_split(edge_index.astype(jnp.int32))
    src2d = jnp.reshape(src1d, (ER, CH))
    dst2d = jnp.reshape(dst1d, (ER, CH))
    xp = jnp.reshape(jnp.pad(x, ((0, NP - N), (0, 0))), (NP // 4, 4 * D))
    eye4 = jnp.eye(4, dtype=jnp.float32)
    W1BD = jnp.kron(eye4, jnp.pad(W1, ((0, 0), (0, 32 - H))))
    W2BD = jnp.kron(eye4, jnp.pad(W2, ((0, 32 - H), (0, 32 - C))))
    b1p = jnp.tile(jnp.pad(b1, (0, 32 - H)), 4).reshape(1, 128)
    b2p = jnp.tile(jnp.pad(b2, (0, 32 - C)), 4).reshape(1, 128)

    deg2 = _sc_degree(dst2d)
    deg2r = jnp.reshape(deg2, (NC, NP // 4, 128))
    hs1p = _tc1(xp, W1BD, deg2r)
    agg1 = _sc_aggregate(jnp.reshape(hs1p, (NP, 32)), src2d, dst2d, 32)
    agg1r = jnp.reshape(agg1, (NC, NP // 4, 128))
    hs2p = _tc2(agg1r, hs1p, deg2r, W2BD, b1p)
    agg2 = _sc_aggregate(jnp.reshape(hs2p, (NP, 32)), src2d, dst2d, 32)
    agg2r = jnp.reshape(agg2, (NC, NP // 4, 128))
    outp = _tc3(agg2r, hs2p, deg2r, b2p)
    return jnp.reshape(outp, (NP, 32))[:N, :C]
